# baseline (speedup 1.0000x reference)
import functools

import jax
import jax.numpy as jnp
from jax import lax
from jax.experimental import pallas as pl
from jax.experimental.pallas import tpu as pltpu
from jax.experimental.pallas import tpu_sc as plsc

_NC, _NS = 2, 16
_NW = _NC * _NS
_CHUNK_ROWS = 16


def _sc_copy(rows, C, in_hbm, out_hbm, bufs, in_sems, out_sems):
    rows_w = rows // _NW
    n_chunks = rows_w // _CHUNK_ROWS
    wid = lax.axis_index("s") * _NC + lax.axis_index("c")
    base = wid * rows_w

    def copy_in(i):
        return pltpu.make_async_copy(
            in_hbm.at[pl.ds(base + i * _CHUNK_ROWS, _CHUNK_ROWS)],
            bufs.at[i % 2],
            in_sems.at[i % 2],
        )

    def copy_out(i):
        return pltpu.make_async_copy(
            bufs.at[i % 2],
            out_hbm.at[pl.ds(base + i * _CHUNK_ROWS, _CHUNK_ROWS)],
            out_sems.at[i % 2],
        )

    copy_in(0).start()
    copy_in(1).start()
    for i in range(n_chunks):
        copy_in(i).wait()
        copy_out(i).start()
        j = i + 2
        if j < n_chunks:
            copy_out(j - 2).wait()
            copy_in(j).start()
    copy_out(n_chunks - 2).wait()
    copy_out(n_chunks - 1).wait()


def kernel(hidden_states):
    B, S, C = hidden_states.shape
    rows = B * S
    x = hidden_states.reshape(rows, C)
    mesh = plsc.VectorSubcoreMesh(core_axis_name="c", subcore_axis_name="s")
    k = functools.partial(
        pl.kernel,
        mesh=mesh,
        out_type=jax.ShapeDtypeStruct((rows, C), hidden_states.dtype),
        scratch_types=[
            pltpu.VMEM((2, _CHUNK_ROWS, C), hidden_states.dtype),
            pltpu.SemaphoreType.DMA((2,)),
            pltpu.SemaphoreType.DMA((2,)),
        ],
    )(functools.partial(_sc_copy, rows, C))
    out = k(x)
    return out.reshape(B, S, C)


# speedup vs baseline: 1.6080x; 1.6080x over previous
import jax
import jax.numpy as jnp
from jax.experimental import pallas as pl
from jax.experimental.pallas import tpu as pltpu

_N_BUF = 8
_CHUNK_ROWS = 512


def _dma_pipeline(in_ref, out_ref, bufs, in_sems, out_sems):
    rows = in_ref.shape[0]
    n_chunks = rows // _CHUNK_ROWS

    def copy_in(i):
        return pltpu.make_async_copy(
            in_ref.at[pl.ds(i * _CHUNK_ROWS, _CHUNK_ROWS)],
            bufs.at[i % _N_BUF],
            in_sems.at[i % _N_BUF],
        )

    def copy_out(i):
        return pltpu.make_async_copy(
            bufs.at[i % _N_BUF],
            out_ref.at[pl.ds(i * _CHUNK_ROWS, _CHUNK_ROWS)],
            out_sems.at[i % _N_BUF],
        )

    for i in range(min(_N_BUF, n_chunks)):
        copy_in(i).start()
    for i in range(n_chunks):
        copy_in(i).wait()
        copy_out(i).start()
        j = i + _N_BUF
        if j < n_chunks:
            copy_out(j - _N_BUF).wait()
            copy_in(j).start()
    for i in range(max(0, n_chunks - _N_BUF), n_chunks):
        copy_out(i).wait()


def kernel(hidden_states):
    B, S, C = hidden_states.shape
    rows = B * S
    x = hidden_states.reshape(rows, C)
    out = pl.pallas_call(
        _dma_pipeline,
        in_specs=[pl.BlockSpec(memory_space=pl.ANY)],
        out_specs=pl.BlockSpec(memory_space=pl.ANY),
        out_shape=jax.ShapeDtypeStruct((rows, C), hidden_states.dtype),
        scratch_shapes=[
            pltpu.VMEM((_N_BUF, _CHUNK_ROWS, C), hidden_states.dtype),
            pltpu.SemaphoreType.DMA((_N_BUF,)),
            pltpu.SemaphoreType.DMA((_N_BUF,)),
        ],
    )(x)
    return out.reshape(B, S, C)


# speedup vs baseline: 1.6163x; 1.0051x over previous
import jax
import jax.numpy as jnp
from jax.experimental import pallas as pl
from jax.experimental.pallas import tpu as pltpu

_N_BUF = 16
_CHUNK_ROWS = 256


def _dma_pipeline(in_ref, out_ref, bufs, in_sems, out_sems):
    rows = in_ref.shape[0]
    n_chunks = rows // _CHUNK_ROWS

    def copy_in(i):
        return pltpu.make_async_copy(
            in_ref.at[pl.ds(i * _CHUNK_ROWS, _CHUNK_ROWS)],
            bufs.at[i % _N_BUF],
            in_sems.at[i % _N_BUF],
        )

    def copy_out(i):
        return pltpu.make_async_copy(
            bufs.at[i % _N_BUF],
            out_ref.at[pl.ds(i * _CHUNK_ROWS, _CHUNK_ROWS)],
            out_sems.at[i % _N_BUF],
        )

    for i in range(min(_N_BUF, n_chunks)):
        copy_in(i).start()
    for i in range(n_chunks):
        copy_in(i).wait()
        copy_out(i).start()
        j = i + _N_BUF
        if j < n_chunks:
            copy_out(j - _N_BUF).wait()
            copy_in(j).start()
    for i in range(max(0, n_chunks - _N_BUF), n_chunks):
        copy_out(i).wait()


def kernel(hidden_states):
    B, S, C = hidden_states.shape
    rows = B * S
    x = hidden_states.reshape(rows, C)
    out = pl.pallas_call(
        _dma_pipeline,
        in_specs=[pl.BlockSpec(memory_space=pl.ANY)],
        out_specs=pl.BlockSpec(memory_space=pl.ANY),
        out_shape=jax.ShapeDtypeStruct((rows, C), hidden_states.dtype),
        scratch_shapes=[
            pltpu.VMEM((_N_BUF, _CHUNK_ROWS, C), hidden_states.dtype),
            pltpu.SemaphoreType.DMA((_N_BUF,)),
            pltpu.SemaphoreType.DMA((_N_BUF,)),
        ],
    )(x)
    return out.reshape(B, S, C)
